# baseline (speedup 1.0000x reference)
import jax
import jax.numpy as jnp
from jax import lax
from jax.experimental import pallas as pl
from jax.experimental.pallas import tpu as pltpu
from jax.experimental.pallas import tpu_sc as plsc

_BATCH = 16384
_HIST = 50
_DIM = 64
_B_TOT = _BATCH * _HIST
_NC = 2
_NS = 16
_NW = _NC * _NS
_BAT_W = _BATCH // _NW
_HH = 2
_NLOOK = 128 * _HH
_NHC = _HIST // _HH
_NCHUNK = 4 * _NHC
_RHO_HI = _BATCH // 128


def _emb_body(table, idx_hbm, out5, idx_v, gg0, gg1, tt0, tt1, ic0, ic1,
              gsem0, gsem1, ssem0, ssem1):
    G = (gg0, gg1)
    T = (tt0, tt1)
    IC = (ic0, ic1)
    gsem = (gsem0, gsem1)
    ssem = (ssem0, ssem1)
    wid = lax.axis_index("s") * _NC + lax.axis_index("c")
    bhi0 = wid * 4

    pltpu.sync_copy(idx_hbm.at[pl.ds(wid * _BAT_W * _HIST, _BAT_W * _HIST)],
                    idx_v)

    iota = lax.iota(jnp.int32, 16)
    iota50 = iota * _HIST
    rowvecs = [iota + (hh * 128 + g * 16)
               for hh in range(_HH) for g in range(8)]

    def build_and_gather(i, b):
        bl = i // _NHC
        hc = i % _NHC
        for hh in range(_HH):
            for g in range(8):
                base = (bl * 128 + g * 16) * _HIST + hc * _HH + hh
                vals = plsc.load_gather(idx_v, [iota50 + base])
                IC[b][pl.ds(hh * 128 + g * 16, 16)] = vals
        pltpu.async_copy(table.at[IC[b]], G[b], gsem[b])

    def process(i, b, skip_store_wait):
        bl = i // _NHC
        hc = i % _NHC
        pltpu.make_async_copy(table.at[IC[b]], G[b], gsem[b]).wait()
        if not skip_store_wait:
            pltpu.make_async_copy(out5.at[pl.ds(0, 16), 0], T[b],
                                  ssem[b]).wait()

        @plsc.parallel_loop(0, _DIM * 8, unroll=4)
        def dbody(i):
            d = i // 8
            g = i % 8
            dvec = jnp.full((16,), 0, jnp.int32) + d
            rv = iota + g * 16
            p = d // 8
            q = d % 8
            v0 = plsc.load_gather(G[b], [rv, dvec])
            T[b][p, q, pl.ds(g * 16, 16)] = v0
            v1 = plsc.load_gather(G[b], [rv + 128, dvec])
            T[b][8 + p, q, pl.ds(g * 16, 16)] = v1
        pltpu.async_copy(T[b], out5.at[pl.ds(hc * 16, 16), bhi0 + bl],
                         ssem[b])

    build_and_gather(0, 0)
    build_and_gather(1, 1)
    process(0, 0, skip_store_wait=True)
    build_and_gather(2, 0)
    process(1, 1, skip_store_wait=True)
    build_and_gather(3, 1)

    def outer(j, carry):
        i0 = 2 * j
        process(i0, 0, skip_store_wait=False)
        build_and_gather(i0 + 2, 0)
        process(i0 + 1, 1, skip_store_wait=False)
        build_and_gather(i0 + 3, 1)
        return carry

    lax.fori_loop(1, _NCHUNK // 2 - 1, outer, 0)
    process(_NCHUNK - 2, 0, skip_store_wait=False)
    process(_NCHUNK - 1, 1, skip_store_wait=False)
    for b in range(2):
        pltpu.make_async_copy(out5.at[pl.ds(0, 16), 0], T[b], ssem[b]).wait()


def kernel(token_ids, param):
    idx = token_ids.reshape(_B_TOT).astype(jnp.int32)
    mesh = plsc.VectorSubcoreMesh(core_axis_name="c", subcore_axis_name="s")
    out5 = pl.kernel(
        _emb_body,
        out_type=jax.ShapeDtypeStruct((_HIST * _DIM // 8, 128, 8, 128),
                                      jnp.float32),
        mesh=mesh,
        compiler_params=pltpu.CompilerParams(use_tc_tiling_on_sc=False,
                                             needs_layout_passes=False),
        scratch_types=[
            pltpu.VMEM((_BAT_W * _HIST,), jnp.int32),
            pltpu.VMEM((_NLOOK, _DIM), jnp.float32),
            pltpu.VMEM((_NLOOK, _DIM), jnp.float32),
            pltpu.VMEM((16, 8, 128), jnp.float32),
            pltpu.VMEM((16, 8, 128), jnp.float32),
            pltpu.VMEM((_NLOOK,), jnp.int32),
            pltpu.VMEM((_NLOOK,), jnp.int32),
            pltpu.SemaphoreType.DMA,
            pltpu.SemaphoreType.DMA,
            pltpu.SemaphoreType.DMA,
            pltpu.SemaphoreType.DMA,
        ],
    )(param, idx)
    return out5.transpose(1, 3, 0, 2).reshape(_BATCH, _HIST, _DIM)


# speedup vs baseline: 1.0277x; 1.0277x over previous
import jax
import jax.numpy as jnp
from jax import lax
from jax.experimental import pallas as pl
from jax.experimental.pallas import tpu as pltpu
from jax.experimental.pallas import tpu_sc as plsc

_BATCH = 16384
_HIST = 50
_DIM = 64
_B_TOT = _BATCH * _HIST
_NC = 2
_NS = 16
_NW = _NC * _NS
_BAT_W = _BATCH // _NW
_HH = 2
_NLOOK = 128 * _HH
_NHC = _HIST // _HH
_NCHUNK = 4 * _NHC
_RHO_HI = _BATCH // 128


def _emb_body(table, idx_hbm, out5, idx_v, gg0, gg1, tt0, tt1, ic0, ic1,
              gsem0, gsem1, ssem0, ssem1):
    G = (gg0, gg1)
    T = (tt0, tt1)
    IC = (ic0, ic1)
    gsem = (gsem0, gsem1)
    ssem = (ssem0, ssem1)
    wid = lax.axis_index("s") * _NC + lax.axis_index("c")
    bhi0 = wid * 4

    pltpu.sync_copy(idx_hbm.at[pl.ds(wid * _BAT_W * _HIST, _BAT_W * _HIST)],
                    idx_v)

    iota = lax.iota(jnp.int32, 16)
    iota50 = iota * _HIST
    rowvecs = [iota + (hh * 128 + g * 16)
               for hh in range(_HH) for g in range(8)]

    def build_and_gather(i, b):
        bl = i // _NHC
        hc = i % _NHC
        for hh in range(_HH):
            for g in range(8):
                base = (bl * 128 + g * 16) * _HIST + hc * _HH + hh
                vals = plsc.load_gather(idx_v, [iota50 + base])
                IC[b][pl.ds(hh * 128 + g * 16, 16)] = vals
        pltpu.async_copy(table.at[IC[b]], G[b], gsem[b])

    def process(i, b, skip_store_wait):
        bl = i // _NHC
        hc = i % _NHC
        pltpu.make_async_copy(table.at[IC[b]], G[b], gsem[b]).wait()
        if not skip_store_wait:
            pltpu.make_async_copy(out5.at[pl.ds(0, 16), 0], T[b],
                                  ssem[b]).wait()

        @plsc.parallel_loop(0, _DIM, unroll=2)
        def dbody(d):
            dvec = jnp.full((16,), 0, jnp.int32) + d
            p = d // 8
            q = d % 8

            @plsc.parallel_loop(0, 8, unroll=8)
            def gbody(g):
                rv = iota + g * 16
                v0 = plsc.load_gather(G[b], [rv, dvec])
                T[b][p, q, pl.ds(g * 16, 16)] = v0
                v1 = plsc.load_gather(G[b], [rv + 128, dvec])
                T[b][8 + p, q, pl.ds(g * 16, 16)] = v1
        pltpu.async_copy(T[b], out5.at[pl.ds(hc * 16, 16), bhi0 + bl],
                         ssem[b])

    build_and_gather(0, 0)
    build_and_gather(1, 1)
    process(0, 0, skip_store_wait=True)
    build_and_gather(2, 0)
    process(1, 1, skip_store_wait=True)
    build_and_gather(3, 1)

    def outer(j, carry):
        i0 = 2 * j
        process(i0, 0, skip_store_wait=False)
        build_and_gather(i0 + 2, 0)
        process(i0 + 1, 1, skip_store_wait=False)
        build_and_gather(i0 + 3, 1)
        return carry

    lax.fori_loop(1, _NCHUNK // 2 - 1, outer, 0)
    process(_NCHUNK - 2, 0, skip_store_wait=False)
    process(_NCHUNK - 1, 1, skip_store_wait=False)
    for b in range(2):
        pltpu.make_async_copy(out5.at[pl.ds(0, 16), 0], T[b], ssem[b]).wait()


def kernel(token_ids, param):
    idx = token_ids.reshape(_B_TOT).astype(jnp.int32)
    mesh = plsc.VectorSubcoreMesh(core_axis_name="c", subcore_axis_name="s")
    out5 = pl.kernel(
        _emb_body,
        out_type=jax.ShapeDtypeStruct((_HIST * _DIM // 8, 128, 8, 128),
                                      jnp.float32),
        mesh=mesh,
        compiler_params=pltpu.CompilerParams(use_tc_tiling_on_sc=False,
                                             needs_layout_passes=False),
        scratch_types=[
            pltpu.VMEM((_BAT_W * _HIST,), jnp.int32),
            pltpu.VMEM((_NLOOK, _DIM), jnp.float32),
            pltpu.VMEM((_NLOOK, _DIM), jnp.float32),
            pltpu.VMEM((16, 8, 128), jnp.float32),
            pltpu.VMEM((16, 8, 128), jnp.float32),
            pltpu.VMEM((_NLOOK,), jnp.int32),
            pltpu.VMEM((_NLOOK,), jnp.int32),
            pltpu.SemaphoreType.DMA,
            pltpu.SemaphoreType.DMA,
            pltpu.SemaphoreType.DMA,
            pltpu.SemaphoreType.DMA,
        ],
    )(param, idx)
    return out5.transpose(1, 3, 0, 2).reshape(_BATCH, _HIST, _DIM)


# speedup vs baseline: 1.1473x; 1.1165x over previous
import jax
import jax.numpy as jnp
from jax import lax
from jax.experimental import pallas as pl
from jax.experimental.pallas import tpu as pltpu
from jax.experimental.pallas import tpu_sc as plsc

_BATCH = 16384
_HIST = 50
_DIM = 64
_B_TOT = _BATCH * _HIST
_NC = 2
_NS = 16
_NW = _NC * _NS
_BAT_W = _BATCH // _NW
_HH = 2
_NLOOK = 128 * _HH
_NHC = _HIST // _HH
_NCHUNK = 4 * _NHC
_RHO_HI = _BATCH // 128


def _emb_body(table, idx_hbm, out5, idx_v, gg0, gg1, tt0, tt1, ic0, ic1,
              gsem0, gsem1, ssem0, ssem1):
    G = (gg0, gg1)
    T = (tt0, tt1)
    IC = (ic0, ic1)
    gsem = (gsem0, gsem1)
    ssem = (ssem0, ssem1)
    wid = lax.axis_index("s") * _NC + lax.axis_index("c")
    bhi0 = wid * 4

    pltpu.sync_copy(idx_hbm.at[pl.ds(wid * _BAT_W * _HIST, _BAT_W * _HIST)],
                    idx_v)

    iota = lax.iota(jnp.int32, 16)
    iota50 = iota * _HIST
    qvec = lax.rem(iota, 8)
    pbase = [2 * c + iota // 8 for c in range(4)]

    def build_and_gather(i, b):
        bl = i // _NHC
        hc = i % _NHC
        for hh in range(_HH):
            for g in range(8):
                base = (bl * 128 + g * 16) * _HIST + hc * _HH + hh
                vals = plsc.load_gather(idx_v, [iota50 + base])
                IC[b][pl.ds(hh * 128 + g * 16, 16)] = vals
        pltpu.async_copy(table.at[IC[b]], G[b], gsem[b])

    def process(i, b, skip_store_wait):
        bl = i // _NHC
        hc = i % _NHC
        pltpu.make_async_copy(table.at[IC[b]], G[b], gsem[b]).wait()
        if not skip_store_wait:
            pltpu.make_async_copy(out5.at[pl.ds(0, 16), 0], T[b],
                                  ssem[b]).wait()

        @plsc.parallel_loop(0, _NLOOK, unroll=4)
        def jbody(j):
            hh8 = (j // 128) * 8
            blo = j % 128
            lvec = jnp.full((16,), 0, jnp.int32) + blo
            for c in range(4):
                v = G[b][j, pl.ds(c * 16, 16)]
                plsc.store_scatter(T[b], [pbase[c] + hh8, qvec, lvec], v)
        pltpu.async_copy(T[b], out5.at[pl.ds(hc * 16, 16), bhi0 + bl],
                         ssem[b])

    build_and_gather(0, 0)
    build_and_gather(1, 1)
    process(0, 0, skip_store_wait=True)
    build_and_gather(2, 0)
    process(1, 1, skip_store_wait=True)
    build_and_gather(3, 1)

    def outer(j, carry):
        i0 = 2 * j
        process(i0, 0, skip_store_wait=False)
        build_and_gather(i0 + 2, 0)
        process(i0 + 1, 1, skip_store_wait=False)
        build_and_gather(i0 + 3, 1)
        return carry

    lax.fori_loop(1, _NCHUNK // 2 - 1, outer, 0)
    process(_NCHUNK - 2, 0, skip_store_wait=False)
    process(_NCHUNK - 1, 1, skip_store_wait=False)
    for b in range(2):
        pltpu.make_async_copy(out5.at[pl.ds(0, 16), 0], T[b], ssem[b]).wait()


def kernel(token_ids, param):
    idx = token_ids.reshape(_B_TOT).astype(jnp.int32)
    mesh = plsc.VectorSubcoreMesh(core_axis_name="c", subcore_axis_name="s")
    out5 = pl.kernel(
        _emb_body,
        out_type=jax.ShapeDtypeStruct((_HIST * _DIM // 8, 128, 8, 128),
                                      jnp.float32),
        mesh=mesh,
        compiler_params=pltpu.CompilerParams(use_tc_tiling_on_sc=False,
                                             needs_layout_passes=False),
        scratch_types=[
            pltpu.VMEM((_BAT_W * _HIST,), jnp.int32),
            pltpu.VMEM((_NLOOK, _DIM), jnp.float32),
            pltpu.VMEM((_NLOOK, _DIM), jnp.float32),
            pltpu.VMEM((16, 8, 128), jnp.float32),
            pltpu.VMEM((16, 8, 128), jnp.float32),
            pltpu.VMEM((_NLOOK,), jnp.int32),
            pltpu.VMEM((_NLOOK,), jnp.int32),
            pltpu.SemaphoreType.DMA,
            pltpu.SemaphoreType.DMA,
            pltpu.SemaphoreType.DMA,
            pltpu.SemaphoreType.DMA,
        ],
    )(param, idx)
    return out5.transpose(1, 3, 0, 2).reshape(_BATCH, _HIST, _DIM)


# speedup vs baseline: 1.2021x; 1.0477x over previous
import jax
import jax.numpy as jnp
from jax import lax
from jax.experimental import pallas as pl
from jax.experimental.pallas import tpu as pltpu
from jax.experimental.pallas import tpu_sc as plsc

_BATCH = 16384
_HIST = 50
_DIM = 64
_B_TOT = _BATCH * _HIST
_NC = 2
_NS = 16
_NW = _NC * _NS
_B_PER_W = _B_TOT // _NW
_CHUNK = 400
_N_CHUNK = _B_PER_W // _CHUNK
_NBUF = 4


def _emb_body(table_hbm, idx_hbm, out_hbm, idx_v, r0, r1, r2, r3,
              g0, g1, g2, g3, s0, s1, s2, s3):
    rows = (r0, r1, r2, r3)
    gsem = (g0, g1, g2, g3)
    ssem = (s0, s1, s2, s3)
    wid = lax.axis_index("s") * _NC + lax.axis_index("c")
    base = wid * _B_PER_W

    pltpu.sync_copy(idx_hbm.at[pl.ds(base, _B_PER_W)], idx_v)

    def start_gather(i, b):
        return pltpu.async_copy(
            table_hbm.at[idx_v.at[pl.ds(i * _CHUNK, _CHUNK)]], rows[b], gsem[b])

    for b in range(_NBUF):
        start_gather(b, b)

    def step(i, b, last):
        pltpu.make_async_copy(
            table_hbm.at[idx_v.at[pl.ds(0, _CHUNK)]], rows[b], gsem[b]).wait()
        st = pltpu.async_copy(
            rows[b], out_hbm.at[pl.ds(base + i * _CHUNK, _CHUNK)], ssem[b])
        if not last:
            st.wait()
            start_gather(i + _NBUF, b)
        return st

    def outer(j, carry):
        for b in range(_NBUF):
            step(j * _NBUF + b, b, last=False)
        return carry

    lax.fori_loop(0, _N_CHUNK // _NBUF - 1, outer, 0)
    tail = []
    for b in range(_NBUF):
        tail.append(step(_N_CHUNK - _NBUF + b, b, last=True))
    for st in tail:
        st.wait()


def kernel(token_ids, param):
    idx = token_ids.reshape(_B_TOT).astype(jnp.int32)
    mesh = plsc.VectorSubcoreMesh(core_axis_name="c", subcore_axis_name="s")
    out = pl.kernel(
        _emb_body,
        out_type=jax.ShapeDtypeStruct((_B_TOT, _DIM), jnp.float32),
        mesh=mesh,
        compiler_params=pltpu.CompilerParams(use_tc_tiling_on_sc=False),
        scratch_types=[
            pltpu.VMEM((_B_PER_W,), jnp.int32),
            pltpu.VMEM((_CHUNK, _DIM), jnp.float32),
            pltpu.VMEM((_CHUNK, _DIM), jnp.float32),
            pltpu.VMEM((_CHUNK, _DIM), jnp.float32),
            pltpu.VMEM((_CHUNK, _DIM), jnp.float32),
            pltpu.SemaphoreType.DMA,
            pltpu.SemaphoreType.DMA,
            pltpu.SemaphoreType.DMA,
            pltpu.SemaphoreType.DMA,
            pltpu.SemaphoreType.DMA,
            pltpu.SemaphoreType.DMA,
            pltpu.SemaphoreType.DMA,
            pltpu.SemaphoreType.DMA,
        ],
    )(param, idx)
    return out.reshape(_BATCH, _HIST, _DIM)
